# Initial kernel scaffold; baseline (speedup 1.0000x reference)
import jax
import jax.numpy as jnp
from jax.experimental import pallas as pl


def kernel(chars, embedding_weight):
    raise NotImplementedError("write your pallas kernel here")


# speedup vs baseline: 4.5461x; 4.5461x over previous
import functools

import jax
import jax.numpy as jnp
from jax import lax
from jax.experimental import pallas as pl
from jax.experimental.pallas import tpu as pltpu
from jax.experimental.pallas import tpu_sc as plsc

_L = 16


def _bucket(cv):
    one = jnp.int32(1)
    b = jnp.where(cv >= 1, one, jnp.int32(0))
    b = jnp.where(cv >= 11, jnp.int32(2), b)
    b = jnp.where(cv >= 37, jnp.int32(3), b)
    b = jnp.where(cv >= 63, jnp.int32(4), b)
    return b


def _make_sc_lookup(n_total, emb, chunk):
    nc, ns = 2, 16
    nw = nc * ns
    per_w = n_total // nw
    assert n_total % nw == 0 and per_w % chunk == 0
    n_chunks = per_w // chunk
    groups = chunk // _L

    mesh = plsc.VectorSubcoreMesh(
        core_axis_name="c",
        subcore_axis_name="s",
        num_cores=nc,
        num_subcores=ns,
    )

    @functools.partial(
        pl.kernel,
        mesh=mesh,
        out_type=jax.ShapeDtypeStruct((n_total * emb,), jnp.float32),
        compiler_params=pltpu.CompilerParams(needs_layout_passes=False),
        scratch_types=[
            pltpu.VMEM((chunk,), jnp.int32),
            pltpu.VMEM((chunk * emb,), jnp.float32),
            pltpu.VMEM((5 * emb,), jnp.float32),
        ],
    )
    def sc_lookup(chars_hbm, w_hbm, out_hbm, ids_v, rows_v, w_v):
        wid = lax.axis_index("s") * nc + lax.axis_index("c")
        base = wid * per_w
        pltpu.sync_copy(w_hbm, w_v)
        lane = lax.iota(jnp.int32, _L)

        def chunk_body(k, _):
            off = base + k * chunk
            pltpu.sync_copy(chars_hbm.at[pl.ds(off, chunk)], ids_v)

            def group_body(g, _):
                cv = ids_v[pl.ds(g * _L, _L)]
                fb = _bucket(cv) * emb
                obase = g * (_L * emb) + lane * emb
                for e in range(emb):
                    row = plsc.load_gather(w_v, [fb + e])
                    plsc.store_scatter(rows_v, [obase + e], row)
                return 0

            lax.fori_loop(0, groups, group_body, 0, unroll=False)
            pltpu.sync_copy(rows_v, out_hbm.at[pl.ds(off * emb, chunk * emb)])
            return 0

        lax.fori_loop(0, n_chunks, chunk_body, 0, unroll=False)

    return sc_lookup


def kernel(chars, embedding_weight):
    b, s = chars.shape
    n_cls, emb = embedding_weight.shape
    n_total = b * s
    out_flat = _make_sc_lookup(n_total, emb, chunk=2048)(
        chars.reshape(n_total), embedding_weight.reshape(n_cls * emb)
    )
    return out_flat.reshape(b, s, emb)
